# baseline (speedup 1.0000x reference)
import functools

import jax
import jax.numpy as jnp
from jax import lax
from jax.experimental import pallas as pl
from jax.experimental.pallas import tpu as pltpu
from jax.experimental.pallas import tpu_sc as plsc

_NC = 2
_NS = 16
_L = 16
_NW = _NC * _NS

_CTX = 20
_NNEG = 20
_D = 32
_RPB = _CTX + _CTX * _NNEG
_RPAD = 432
_GCHUNKS = (112, 112, 112, 96)
_GOFFS = (0, 112, 224, 336)
_NGRP = _RPAD // _L
_PERM = (0, 8, 4, 12, 2, 10, 6, 14, 1, 9, 5, 13, 3, 11, 7, 15)


@functools.lru_cache(maxsize=2)
def _build_sc_scores(batch):
    bpw = batch // _NW
    mesh = plsc.VectorSubcoreMesh(core_axis_name="c", subcore_axis_name="s")

    @functools.partial(
        pl.kernel,
        mesh=mesh,
        out_type=jax.ShapeDtypeStruct((batch, _RPAD), jnp.float32),
        scratch_types=[
            pltpu.VMEM((bpw, _D), jnp.float32),
            pltpu.VMEM((bpw, _RPAD), jnp.int32),
            pltpu.VMEM((4, _RPAD, 2, _L), jnp.bfloat16),
            pltpu.VMEM((_RPAD,), jnp.float32),
            pltpu.SemaphoreType.DMA,
            pltpu.SemaphoreType.DMA,
            pltpu.SemaphoreType.DMA,
            pltpu.SemaphoreType.DMA,
        ],
        compiler_params=pltpu.CompilerParams(use_tc_tiling_on_sc=False),
    )
    def sc_scores(wo_hbm, ivecs_hbm, idx_hbm, out_hbm,
                  ivec_v, idx_v, rows_v, sc_v, sem0, sem1, sem2, sem3):
        sems = (sem0, sem1, sem2, sem3)
        wid = lax.axis_index("s") * _NC + lax.axis_index("c")
        base = wid * bpw

        pltpu.sync_copy(ivecs_hbm.at[pl.ds(base, bpw)], ivec_v)
        pltpu.sync_copy(idx_hbm.at[pl.ds(base, bpw)], idx_v)

        lanes = lax.iota(jnp.int32, _L)
        perms = {k: lanes ^ k for k in (8, 4, 2, 1)}
        sels = {k: (lanes & k) == 0 for k in (8, 4, 2, 1)}

        def fire(b, buf, sem):
            return [
                pltpu.async_copy(
                    wo_hbm.at[idx_v.at[b, pl.ds(off, sz)]],
                    rows_v.at[buf, pl.ds(off, sz)],
                    sem,
                )
                for off, sz in zip(_GOFFS, _GCHUNKS)
            ]

        def drain(buf, sem):
            pltpu.make_async_copy(
                wo_hbm.at[pl.ds(0, _RPAD)], rows_v.at[buf], sem).wait()

        def compute(b, buf):
            iv0 = ivec_v[b, pl.ds(0, _L)]
            iv1 = ivec_v[b, pl.ds(_L, _L)]

            def per_group(g, c2):
                base_j = g * _L
                ps = []
                for i in range(_L):
                    j = base_j + _PERM[i]
                    x32 = rows_v[buf, j, :, :].astype(jnp.float32)
                    ps.append(x32[0] * iv0 + x32[1] * iv1)
                for k in (8, 4, 2, 1):
                    sel, prm = sels[k], perms[k]
                    nxt = []
                    for a, b2 in zip(ps[0::2], ps[1::2]):
                        c = jnp.where(sel, a, b2)
                        d2 = jnp.where(sel, b2, a)
                        nxt.append(c + jnp.take(d2, prm))
                    ps = nxt
                sc_v[pl.ds(g * _L, _L)] = ps[0]
                return c2

            lax.fori_loop(0, _NGRP, per_group, 0, unroll=3)
            pltpu.sync_copy(sc_v, out_hbm.at[base + b])

        for q in range(3):
            fire(q, q, sems[q])

        def quad_body(it, carry):
            b0 = it * 4
            for q in range(4):
                b = b0 + q
                nxt = b + 3
                nbuf = (q + 3) % 4

                @pl.when(nxt < bpw)
                def _():
                    fire(nxt, nbuf, sems[nbuf])

                drain(q, sems[q])
                compute(b, q)
            return carry

        lax.fori_loop(0, bpw // 4, quad_body, 0)

    return sc_scores


def _loss_block(x_ref, o_ref):
    i = pl.program_id(0)

    @pl.when(i == 0)
    def _():
        o_ref[0, 0] = 0.0

    x = x_ref[...]
    col = lax.broadcasted_iota(jnp.int32, x.shape, 1)
    t = jnp.where(col < _CTX, x, -x)
    v = jnp.log(jax.nn.sigmoid(t))
    v = jnp.where(col < _RPB, v, 0.0)
    o_ref[0, 0] += jnp.sum(v)


@functools.lru_cache(maxsize=2)
def _build_tc_loss(batch, blk):
    return pl.pallas_call(
        _loss_block,
        grid=(batch // blk,),
        in_specs=[pl.BlockSpec((blk, _RPAD), lambda i: (i, 0))],
        out_specs=pl.BlockSpec(
            (1, 1), lambda i: (0, 0), memory_space=pltpu.SMEM),
        out_shape=jax.ShapeDtypeStruct((1, 1), jnp.float32),
    )


def kernel(iword, owords, nwords, Wi, Wo):
    batch = iword.shape[0]
    pad = jnp.zeros((batch, _RPAD - _RPB), jnp.int32)
    idx_all = jnp.concatenate([owords, nwords, pad], axis=1)
    ivecs = jnp.take(Wi, iword, axis=0)
    wo16 = Wo.astype(jnp.bfloat16).reshape(Wo.shape[0], 2, _L)

    scores = _build_sc_scores(batch)(wo16, ivecs, idx_all)
    total = _build_tc_loss(batch, 256)(scores)
    return total[0, 0] * (-1.0 / (batch * _CTX))


# speedup vs baseline: 3.1340x; 3.1340x over previous
import functools

import jax
import jax.numpy as jnp
from jax import lax
from jax.experimental import pallas as pl
from jax.experimental.pallas import tpu as pltpu
from jax.experimental.pallas import tpu_sc as plsc

_NC = 2
_NS = 16
_L = 16
_NW = _NC * _NS

_CTX = 20
_NNEG = 20
_D = 32
_RPB = _CTX + _CTX * _NNEG
_RPAD = 432
_GCHUNKS = (112, 112, 112, 96)
_GOFFS = (0, 112, 224, 336)
_NGRP = _RPAD // _L
_PERM = (0, 8, 4, 12, 2, 10, 6, 14, 1, 9, 5, 13, 3, 11, 7, 15)


@functools.lru_cache(maxsize=2)
def _build_sc_scores(batch):
    bpw = batch // _NW
    mesh = plsc.VectorSubcoreMesh(core_axis_name="c", subcore_axis_name="s")

    @functools.partial(
        pl.kernel,
        mesh=mesh,
        out_type=jax.ShapeDtypeStruct((batch, _RPAD), jnp.float32),
        scratch_types=[
            pltpu.VMEM((bpw, _D), jnp.float32),
            pltpu.VMEM((bpw, _RPAD), jnp.int32),
            pltpu.VMEM((4, _RPAD, _D), jnp.bfloat16),
            pltpu.VMEM((_RPAD,), jnp.float32),
            pltpu.SemaphoreType.DMA,
            pltpu.SemaphoreType.DMA,
            pltpu.SemaphoreType.DMA,
            pltpu.SemaphoreType.DMA,
        ],
        compiler_params=pltpu.CompilerParams(use_tc_tiling_on_sc=False),
    )
    def sc_scores(wo_hbm, ivecs_hbm, idx_hbm, out_hbm,
                  ivec_v, idx_v, rows_v, sc_v, sem0, sem1, sem2, sem3):
        sems = (sem0, sem1, sem2, sem3)
        wid = lax.axis_index("s") * _NC + lax.axis_index("c")
        base = wid * bpw

        pltpu.sync_copy(ivecs_hbm.at[pl.ds(base, bpw)], ivec_v)
        pltpu.sync_copy(idx_hbm.at[pl.ds(base, bpw)], idx_v)

        lanes = lax.iota(jnp.int32, _L)
        perms = {k: lanes ^ k for k in (8, 4, 2, 1)}
        sels = {k: (lanes & k) == 0 for k in (8, 4, 2, 1)}

        def fire(b, buf, sem):
            return [
                pltpu.async_copy(
                    wo_hbm.at[idx_v.at[b, pl.ds(off, sz)]],
                    rows_v.at[buf, pl.ds(off, sz)],
                    sem,
                )
                for off, sz in zip(_GOFFS, _GCHUNKS)
            ]

        def drain(buf, sem):
            pltpu.make_async_copy(
                wo_hbm.at[pl.ds(0, _RPAD)], rows_v.at[buf], sem).wait()

        def compute(b, buf):
            iv0 = ivec_v[b, pl.ds(0, _L)]
            iv1 = ivec_v[b, pl.ds(_L, _L)]

            def per_group(g, c2):
                base_j = g * _L
                ps = []
                for i in range(_L):
                    j = base_j + _PERM[i]
                    x32 = rows_v[buf, j, :].reshape(2, _L).astype(jnp.float32)
                    ps.append(x32[0] * iv0 + x32[1] * iv1)
                for k in (8, 4, 2, 1):
                    sel, prm = sels[k], perms[k]
                    nxt = []
                    for a, b2 in zip(ps[0::2], ps[1::2]):
                        c = jnp.where(sel, a, b2)
                        d2 = jnp.where(sel, b2, a)
                        nxt.append(c + jnp.take(d2, prm))
                    ps = nxt
                sc_v[pl.ds(g * _L, _L)] = ps[0]
                return c2

            lax.fori_loop(0, _NGRP, per_group, 0, unroll=3)
            pltpu.sync_copy(sc_v, out_hbm.at[base + b])

        for q in range(3):
            fire(q, q, sems[q])

        def quad_body(it, carry):
            b0 = it * 4
            for q in range(4):
                b = b0 + q
                nxt = b + 3
                nbuf = (q + 3) % 4

                @pl.when(nxt < bpw)
                def _():
                    fire(nxt, nbuf, sems[nbuf])

                drain(q, sems[q])
                compute(b, q)
            return carry

        lax.fori_loop(0, bpw // 4, quad_body, 0)

    return sc_scores


def _loss_block(x_ref, o_ref):
    i = pl.program_id(0)

    @pl.when(i == 0)
    def _():
        o_ref[0, 0] = 0.0

    x = x_ref[...]
    col = lax.broadcasted_iota(jnp.int32, x.shape, 1)
    t = jnp.where(col < _CTX, x, -x)
    v = jnp.log(jax.nn.sigmoid(t))
    v = jnp.where(col < _RPB, v, 0.0)
    o_ref[0, 0] += jnp.sum(v)


@functools.lru_cache(maxsize=2)
def _build_tc_loss(batch, blk):
    return pl.pallas_call(
        _loss_block,
        grid=(batch // blk,),
        in_specs=[pl.BlockSpec((blk, _RPAD), lambda i: (i, 0))],
        out_specs=pl.BlockSpec(
            (1, 1), lambda i: (0, 0), memory_space=pltpu.SMEM),
        out_shape=jax.ShapeDtypeStruct((1, 1), jnp.float32),
    )


def kernel(iword, owords, nwords, Wi, Wo):
    batch = iword.shape[0]
    pad = jnp.zeros((batch, _RPAD - _RPB), jnp.int32)
    idx_all = jnp.concatenate([owords, nwords, pad], axis=1)
    ivecs = jnp.take(Wi, iword, axis=0)
    wo16 = Wo.astype(jnp.bfloat16)

    scores = _build_sc_scores(batch)(wo16, ivecs, idx_all)
    total = _build_tc_loss(batch, 256)(scores)
    return total[0, 0] * (-1.0 / (batch * _CTX))
